# baseline (speedup 1.0000x reference)
import functools

import jax
import jax.numpy as jnp
from jax import lax
from jax.experimental import pallas as pl
from jax.experimental.pallas import tpu as pltpu
from jax.experimental.pallas import tpu_sc as plsc

N = 10000
NP = 10240
E = 320000
EP = 327680
C1 = 80
C2 = 160
B = 16
NB1 = C1 // B
NB2 = C2 // B
NC = 2
NS = 16
RPW = NP // NS

_mesh = plsc.VectorSubcoreMesh(core_axis_name="c", subcore_axis_name="s")


@functools.partial(
    pl.kernel,
    out_type=jax.ShapeDtypeStruct((NC, NP, 16), jnp.float32),
    mesh=_mesh,
    scratch_types=[
        pltpu.VMEM((C1, 128), jnp.int32),
        pltpu.VMEM((128, 16), jnp.float32),
        pltpu.VMEM((128, 16), jnp.float32),
        pltpu.VMEM_SHARED((NP, 16), jnp.float32),
    ],
)
def _sc_deg(dst_hbm, out_hbm, didx, ones_rows, zbuf, acc):
    c = lax.axis_index("c")
    s = lax.axis_index("s")
    wid = c * NS + s
    pltpu.sync_copy(dst_hbm.at[wid], didx)
    zero = jnp.zeros((16,), jnp.float32)
    ones = jnp.ones((16,), jnp.float32)

    def init(i, carry):
        zbuf[i, :] = zero
        ones_rows[i, :] = ones
        return carry

    lax.fori_loop(0, 128, init, 0)
    base = s * RPW

    def za(k, carry):
        pltpu.sync_copy(zbuf, acc.at[pl.ds(base + k * 128, 128)])
        return carry

    lax.fori_loop(0, RPW // 128, za, 0)
    plsc.subcore_barrier()

    def body(j, carry):
        pltpu.sync_copy(ones_rows, acc.at[didx.at[j]], add=True)
        return carry

    lax.fori_loop(0, C1, body, 0)
    plsc.subcore_barrier()
    pltpu.sync_copy(acc.at[pl.ds(base, RPW)], out_hbm.at[c, pl.ds(base, RPW)])


def _zero_acc(rows, acc, s):
    zero = jnp.zeros((16,), jnp.float32)

    def zb(i, carry):
        rows[i // 8, pl.ds((i % 8) * 16, 16)] = zero
        return carry

    lax.fori_loop(0, 1024, zb, 0)
    base = s * RPW

    def za(k, carry):
        pltpu.sync_copy(rows, acc.at[pl.ds(base + k * 128, 128)])
        return carry

    lax.fori_loop(0, RPW // 128, za, 0)


def _spmm_loop(n_blocks, in_hbm, src_hbm, dst_hbm, w, sidx, didx, rows0, rows1, acc, sem0, sem1):
    bufs = (rows0, rows1)
    sems = (sem0, sem1)

    def blk(bi, carry):
        pltpu.sync_copy(src_hbm.at[w, bi], sidx)
        pltpu.sync_copy(dst_hbm.at[w, bi], didx)
        cps = [None, None]
        cps[0] = pltpu.async_copy(in_hbm.at[sidx.at[0]], bufs[0], sems[0])
        for j in range(B):
            cps[j % 2].wait()
            if j + 1 < B:
                nb = (j + 1) % 2
                cps[nb] = pltpu.async_copy(in_hbm.at[sidx.at[j + 1]], bufs[nb], sems[nb])
            pltpu.sync_copy(bufs[j % 2], acc.at[didx.at[j]], add=True)
        return carry

    lax.fori_loop(0, n_blocks, blk, 0)


@functools.partial(
    pl.kernel,
    out_type=jax.ShapeDtypeStruct((NC, NP, 128), jnp.float32),
    mesh=_mesh,
    scratch_types=[
        pltpu.VMEM((B, 128), jnp.int32),
        pltpu.VMEM((B, 128), jnp.int32),
        pltpu.VMEM((128, 128), jnp.float32),
        pltpu.VMEM((128, 128), jnp.float32),
        pltpu.VMEM_SHARED((NP, 128), jnp.float32),
        pltpu.SemaphoreType.DMA,
        pltpu.SemaphoreType.DMA,
    ],
)
def _sc_spmm1(xs_hbm, src_hbm, dst_hbm, out_hbm, sidx, didx, rows0, rows1, acc, sem0, sem1):
    c = lax.axis_index("c")
    s = lax.axis_index("s")
    wid = c * NS + s
    _zero_acc(rows0, acc, s)
    plsc.subcore_barrier()
    _spmm_loop(NB1, xs_hbm, src_hbm, dst_hbm, wid, sidx, didx, rows0, rows1, acc, sem0, sem1)
    plsc.subcore_barrier()
    base = s * RPW
    pltpu.sync_copy(acc.at[pl.ds(base, RPW)], out_hbm.at[c, pl.ds(base, RPW)])


@functools.partial(
    pl.kernel,
    out_type=jax.ShapeDtypeStruct((NC, NP, 128), jnp.float32),
    mesh=_mesh,
    scratch_types=[
        pltpu.VMEM((B, 128), jnp.int32),
        pltpu.VMEM((B, 128), jnp.int32),
        pltpu.VMEM((128, 128), jnp.float32),
        pltpu.VMEM((128, 128), jnp.float32),
        pltpu.VMEM_SHARED((NP, 128), jnp.float32),
        pltpu.SemaphoreType.DMA,
        pltpu.SemaphoreType.DMA,
    ],
)
def _sc_spmm2(ysA_hbm, ysB_hbm, src_hbm, dst_hbm, out_hbm, sidx, didx, rows0, rows1, acc, sem0, sem1):
    c = lax.axis_index("c")
    s = lax.axis_index("s")
    _zero_acc(rows0, acc, s)
    plsc.subcore_barrier()

    @pl.when(c == 0)
    def _():
        _spmm_loop(NB2, ysA_hbm, src_hbm, dst_hbm, s, sidx, didx, rows0, rows1, acc, sem0, sem1)

    @pl.when(c == 1)
    def _():
        _spmm_loop(NB2, ysB_hbm, src_hbm, dst_hbm, s, sidx, didx, rows0, rows1, acc, sem0, sem1)

    plsc.subcore_barrier()
    base = s * RPW
    pltpu.sync_copy(acc.at[pl.ds(base, RPW)], out_hbm.at[c, pl.ds(base, RPW)])


_BR = 1024
_G = NP // _BR


def _tc_scale(x_pad, degp):
    def body(x_ref, degp_ref, xs_ref, dinv_ref):
        deg = 1.0 + degp_ref[0, :, 0] + degp_ref[1, :, 0]
        dinv = lax.rsqrt(deg)[:, None]
        xs_ref[...] = x_ref[...] * dinv
        dinv_ref[...] = dinv

    return pl.pallas_call(
        body,
        grid=(_G,),
        in_specs=[
            pl.BlockSpec((_BR, 128), lambda i: (i, 0)),
            pl.BlockSpec((NC, _BR, 16), lambda i: (0, i, 0)),
        ],
        out_specs=[
            pl.BlockSpec((_BR, 128), lambda i: (i, 0)),
            pl.BlockSpec((_BR, 1), lambda i: (i, 0)),
        ],
        out_shape=[
            jax.ShapeDtypeStruct((NP, 128), jnp.float32),
            jax.ShapeDtypeStruct((NP, 1), jnp.float32),
        ],
    )(x_pad, degp)


def _layer_norm_rows(h, g, b):
    mu = jnp.mean(h, axis=-1, keepdims=True)
    var = jnp.mean((h - mu) ** 2, axis=-1, keepdims=True)
    return (h - mu) * lax.rsqrt(var + 1e-5) * g + b


def _tc_layer1(acc1, xs, dinv, W1, b1, ln_g, ln_b):
    def body(acc_ref, xs_ref, dinv_ref, w_ref, b1_ref, g_ref, b_ref, ysA_ref, ysB_ref):
        dinv = dinv_ref[...]
        agg = (acc_ref[0] + acc_ref[1] + xs_ref[...]) * dinv
        h = jnp.dot(agg, w_ref[...], preferred_element_type=jnp.float32) + b1_ref[...]
        h = _layer_norm_rows(h, g_ref[...], b_ref[...])
        h = jnp.maximum(h, 0.0)
        ys = h * dinv
        ysA_ref[...] = ys[:, :128]
        ysB_ref[...] = ys[:, 128:]

    return pl.pallas_call(
        body,
        grid=(_G,),
        in_specs=[
            pl.BlockSpec((NC, _BR, 128), lambda i: (0, i, 0)),
            pl.BlockSpec((_BR, 128), lambda i: (i, 0)),
            pl.BlockSpec((_BR, 1), lambda i: (i, 0)),
            pl.BlockSpec((128, 256), lambda i: (0, 0)),
            pl.BlockSpec((1, 256), lambda i: (0, 0)),
            pl.BlockSpec((1, 256), lambda i: (0, 0)),
            pl.BlockSpec((1, 256), lambda i: (0, 0)),
        ],
        out_specs=[
            pl.BlockSpec((_BR, 128), lambda i: (i, 0)),
            pl.BlockSpec((_BR, 128), lambda i: (i, 0)),
        ],
        out_shape=[
            jax.ShapeDtypeStruct((NP, 128), jnp.float32),
            jax.ShapeDtypeStruct((NP, 128), jnp.float32),
        ],
    )(acc1, xs, dinv, W1, b1, ln_g, ln_b)


def _tc_layer2(acc2, ysA, ysB, dinv, W2, b2, ln_g, ln_b, Wf, bf):
    def body(acc_ref, ysA_ref, ysB_ref, dinv_ref, w2_ref, b2_ref, g_ref, b_ref,
             wf_ref, bf_ref, out_ref):
        dinv = dinv_ref[...]
        agg = jnp.concatenate(
            [acc_ref[0] + ysA_ref[...], acc_ref[1] + ysB_ref[...]], axis=1) * dinv
        h = jnp.dot(agg, w2_ref[...], preferred_element_type=jnp.float32) + b2_ref[...]
        h = _layer_norm_rows(h, g_ref[...], b_ref[...])
        out_ref[...] = (
            jnp.dot(h, wf_ref[...], preferred_element_type=jnp.float32) + bf_ref[...]
        )

    return pl.pallas_call(
        body,
        grid=(_G,),
        in_specs=[
            pl.BlockSpec((NC, _BR, 128), lambda i: (0, i, 0)),
            pl.BlockSpec((_BR, 128), lambda i: (i, 0)),
            pl.BlockSpec((_BR, 128), lambda i: (i, 0)),
            pl.BlockSpec((_BR, 1), lambda i: (i, 0)),
            pl.BlockSpec((256, 256), lambda i: (0, 0)),
            pl.BlockSpec((1, 256), lambda i: (0, 0)),
            pl.BlockSpec((1, 256), lambda i: (0, 0)),
            pl.BlockSpec((1, 256), lambda i: (0, 0)),
            pl.BlockSpec((256, 128), lambda i: (0, 0)),
            pl.BlockSpec((1, 128), lambda i: (0, 0)),
        ],
        out_specs=pl.BlockSpec((_BR, 128), lambda i: (i, 0)),
        out_shape=jax.ShapeDtypeStruct((NP, 128), jnp.float32),
    )(acc2, ysA, ysB, dinv, W2, b2, ln_g, ln_b, Wf, bf)


def kernel(x, edge_index, W1, b1, W2, b2, ln_g, ln_b, Wf, bf):
    src = edge_index[0]
    dst = edge_index[1]
    pad = N + (jnp.arange(EP - E, dtype=jnp.int32) % (NP - N))
    srcp = jnp.concatenate([src, pad])
    dstp = jnp.concatenate([dst, jnp.full((EP - E,), NP - 1, jnp.int32)])
    srcA = srcp.reshape(32, NB1, B, 128)
    dstA = dstp.reshape(32, NB1, B, 128)
    srcB = srcp.reshape(16, NB2, B, 128)
    dstB = dstp.reshape(16, NB2, B, 128)
    x_pad = jnp.pad(x, ((0, NP - N), (0, 0)))

    degp = _sc_deg(dstp.reshape(32, C1, 128))
    xs, dinv = _tc_scale(x_pad, degp)
    acc1 = _sc_spmm1(xs, srcA, dstA)
    b1r = b1.reshape(1, -1)
    gr = ln_g.reshape(1, -1)
    br = ln_b.reshape(1, -1)
    ysA, ysB = _tc_layer1(acc1, xs, dinv, W1, b1r, gr, br)
    acc2 = _sc_spmm2(ysA, ysB, srcB, dstB)
    outp = _tc_layer2(acc2, ysA, ysB, dinv, W2, b2.reshape(1, -1), gr, br,
                      Wf, bf.reshape(1, -1))
    return outp[:N]


# speedup vs baseline: 1.0268x; 1.0268x over previous
import functools

import jax
import jax.numpy as jnp
from jax import lax
from jax.experimental import pallas as pl
from jax.experimental.pallas import tpu as pltpu
from jax.experimental.pallas import tpu_sc as plsc

N = 10000
NP = 10240
E = 320000
EP = 327680
C1 = 80
C2 = 160
B = 16
NB1 = C1 // B
NB2 = C2 // B
NC = 2
NS = 16
RPW = NP // NS

_mesh = plsc.VectorSubcoreMesh(core_axis_name="c", subcore_axis_name="s")


@functools.partial(
    pl.kernel,
    out_type=jax.ShapeDtypeStruct((NC, NP, 16), jnp.float32),
    mesh=_mesh,
    scratch_types=[
        pltpu.VMEM((C1, 128), jnp.int32),
        pltpu.VMEM((128, 16), jnp.float32),
        pltpu.VMEM((128, 16), jnp.float32),
        pltpu.VMEM_SHARED((NP, 16), jnp.float32),
    ],
)
def _sc_deg(dst_hbm, out_hbm, didx, ones_rows, zbuf, acc):
    c = lax.axis_index("c")
    s = lax.axis_index("s")
    wid = c * NS + s
    pltpu.sync_copy(dst_hbm.at[wid], didx)
    zero = jnp.zeros((16,), jnp.float32)
    ones = jnp.ones((16,), jnp.float32)

    def init(i, carry):
        zbuf[i, :] = zero
        ones_rows[i, :] = ones
        return carry

    lax.fori_loop(0, 128, init, 0)
    base = s * RPW

    def za(k, carry):
        pltpu.sync_copy(zbuf, acc.at[pl.ds(base + k * 128, 128)])
        return carry

    lax.fori_loop(0, RPW // 128, za, 0)
    plsc.subcore_barrier()

    def body(j, carry):
        pltpu.sync_copy(ones_rows, acc.at[didx.at[j]], add=True)
        return carry

    lax.fori_loop(0, C1, body, 0)
    plsc.subcore_barrier()
    pltpu.sync_copy(acc.at[pl.ds(base, RPW)], out_hbm.at[c, pl.ds(base, RPW)])


def _zero_acc(rows, acc, s):
    zero = jnp.zeros((16,), jnp.float32)

    def zb(i, carry):
        rows[i // 8, pl.ds((i % 8) * 16, 16)] = zero
        return carry

    lax.fori_loop(0, 1024, zb, 0)
    base = s * RPW

    def za(k, carry):
        pltpu.sync_copy(rows, acc.at[pl.ds(base + k * 128, 128)])
        return carry

    lax.fori_loop(0, RPW // 128, za, 0)


def _spmm_loop(n_blocks, in_hbm, src_hbm, dst_hbm, w, sidx, didx, rows0, rows1,
               acc, sem0, sem1, isems, isemd):
    bufs = (rows0, rows1)
    sems = (sem0, sem1)
    pltpu.sync_copy(src_hbm.at[w, 0], sidx.at[0])
    pltpu.sync_copy(dst_hbm.at[w, 0], didx.at[0])

    def blk(bi, carry):
        cur = lax.rem(bi, 2)
        nxt = 1 - cur
        bnext = jnp.minimum(bi + 1, n_blocks - 1)
        hp_s = pltpu.async_copy(src_hbm.at[w, bnext], sidx.at[nxt], isems)
        hp_d = pltpu.async_copy(dst_hbm.at[w, bnext], didx.at[nxt], isemd)
        cps = [None, None]
        cps[0] = pltpu.async_copy(in_hbm.at[sidx.at[cur, 0]], bufs[0], sems[0])
        for j in range(B):
            cps[j % 2].wait()
            if j + 1 < B:
                nb = (j + 1) % 2
                cps[nb] = pltpu.async_copy(in_hbm.at[sidx.at[cur, j + 1]], bufs[nb], sems[nb])
            pltpu.sync_copy(bufs[j % 2], acc.at[didx.at[cur, j]], add=True)
        hp_s.wait()
        hp_d.wait()
        return carry

    lax.fori_loop(0, n_blocks, blk, 0)


@functools.partial(
    pl.kernel,
    out_type=jax.ShapeDtypeStruct((NC, NP, 128), jnp.float32),
    mesh=_mesh,
    scratch_types=[
        pltpu.VMEM((2, B, 128), jnp.int32),
        pltpu.VMEM((2, B, 128), jnp.int32),
        pltpu.VMEM((128, 128), jnp.float32),
        pltpu.VMEM((128, 128), jnp.float32),
        pltpu.VMEM_SHARED((NP, 128), jnp.float32),
        pltpu.SemaphoreType.DMA,
        pltpu.SemaphoreType.DMA,
        pltpu.SemaphoreType.DMA,
        pltpu.SemaphoreType.DMA,
    ],
)
def _sc_spmm1(xs_hbm, src_hbm, dst_hbm, out_hbm, sidx, didx, rows0, rows1, acc,
              sem0, sem1, isems, isemd):
    c = lax.axis_index("c")
    s = lax.axis_index("s")
    wid = c * NS + s
    _zero_acc(rows0, acc, s)
    plsc.subcore_barrier()
    _spmm_loop(NB1, xs_hbm, src_hbm, dst_hbm, wid, sidx, didx, rows0, rows1, acc,
               sem0, sem1, isems, isemd)
    plsc.subcore_barrier()
    base = s * RPW
    pltpu.sync_copy(acc.at[pl.ds(base, RPW)], out_hbm.at[c, pl.ds(base, RPW)])


@functools.partial(
    pl.kernel,
    out_type=jax.ShapeDtypeStruct((NC, NP, 128), jnp.float32),
    mesh=_mesh,
    scratch_types=[
        pltpu.VMEM((2, B, 128), jnp.int32),
        pltpu.VMEM((2, B, 128), jnp.int32),
        pltpu.VMEM((128, 128), jnp.float32),
        pltpu.VMEM((128, 128), jnp.float32),
        pltpu.VMEM_SHARED((NP, 128), jnp.float32),
        pltpu.SemaphoreType.DMA,
        pltpu.SemaphoreType.DMA,
        pltpu.SemaphoreType.DMA,
        pltpu.SemaphoreType.DMA,
    ],
)
def _sc_spmm2(ysA_hbm, ysB_hbm, src_hbm, dst_hbm, out_hbm, sidx, didx, rows0, rows1,
              acc, sem0, sem1, isems, isemd):
    c = lax.axis_index("c")
    s = lax.axis_index("s")
    _zero_acc(rows0, acc, s)
    plsc.subcore_barrier()

    @pl.when(c == 0)
    def _():
        _spmm_loop(NB2, ysA_hbm, src_hbm, dst_hbm, s, sidx, didx, rows0, rows1, acc,
                   sem0, sem1, isems, isemd)

    @pl.when(c == 1)
    def _():
        _spmm_loop(NB2, ysB_hbm, src_hbm, dst_hbm, s, sidx, didx, rows0, rows1, acc,
                   sem0, sem1, isems, isemd)

    plsc.subcore_barrier()
    base = s * RPW
    pltpu.sync_copy(acc.at[pl.ds(base, RPW)], out_hbm.at[c, pl.ds(base, RPW)])


_BR = 1024
_G = NP // _BR


def _tc_scale(x_pad, degp):
    def body(x_ref, degp_ref, xs_ref, dinv_ref):
        deg = 1.0 + degp_ref[0, :, 0] + degp_ref[1, :, 0]
        dinv = lax.rsqrt(deg)[:, None]
        xs_ref[...] = x_ref[...] * dinv
        dinv_ref[...] = dinv

    return pl.pallas_call(
        body,
        grid=(_G,),
        in_specs=[
            pl.BlockSpec((_BR, 128), lambda i: (i, 0)),
            pl.BlockSpec((NC, _BR, 16), lambda i: (0, i, 0)),
        ],
        out_specs=[
            pl.BlockSpec((_BR, 128), lambda i: (i, 0)),
            pl.BlockSpec((_BR, 1), lambda i: (i, 0)),
        ],
        out_shape=[
            jax.ShapeDtypeStruct((NP, 128), jnp.float32),
            jax.ShapeDtypeStruct((NP, 1), jnp.float32),
        ],
    )(x_pad, degp)


def _layer_norm_rows(h, g, b):
    mu = jnp.mean(h, axis=-1, keepdims=True)
    var = jnp.mean((h - mu) ** 2, axis=-1, keepdims=True)
    return (h - mu) * lax.rsqrt(var + 1e-5) * g + b


def _tc_layer1(acc1, xs, dinv, W1, b1, ln_g, ln_b):
    def body(acc_ref, xs_ref, dinv_ref, w_ref, b1_ref, g_ref, b_ref, ysA_ref, ysB_ref):
        dinv = dinv_ref[...]
        agg = (acc_ref[0] + acc_ref[1] + xs_ref[...]) * dinv
        h = jnp.dot(agg, w_ref[...], preferred_element_type=jnp.float32) + b1_ref[...]
        h = _layer_norm_rows(h, g_ref[...], b_ref[...])
        h = jnp.maximum(h, 0.0)
        ys = h * dinv
        ysA_ref[...] = ys[:, :128]
        ysB_ref[...] = ys[:, 128:]

    return pl.pallas_call(
        body,
        grid=(_G,),
        in_specs=[
            pl.BlockSpec((NC, _BR, 128), lambda i: (0, i, 0)),
            pl.BlockSpec((_BR, 128), lambda i: (i, 0)),
            pl.BlockSpec((_BR, 1), lambda i: (i, 0)),
            pl.BlockSpec((128, 256), lambda i: (0, 0)),
            pl.BlockSpec((1, 256), lambda i: (0, 0)),
            pl.BlockSpec((1, 256), lambda i: (0, 0)),
            pl.BlockSpec((1, 256), lambda i: (0, 0)),
        ],
        out_specs=[
            pl.BlockSpec((_BR, 128), lambda i: (i, 0)),
            pl.BlockSpec((_BR, 128), lambda i: (i, 0)),
        ],
        out_shape=[
            jax.ShapeDtypeStruct((NP, 128), jnp.float32),
            jax.ShapeDtypeStruct((NP, 128), jnp.float32),
        ],
    )(acc1, xs, dinv, W1, b1, ln_g, ln_b)


def _tc_layer2(acc2, ysA, ysB, dinv, W2, b2, ln_g, ln_b, Wf, bf):
    def body(acc_ref, ysA_ref, ysB_ref, dinv_ref, w2_ref, b2_ref, g_ref, b_ref,
             wf_ref, bf_ref, out_ref):
        dinv = dinv_ref[...]
        agg = jnp.concatenate(
            [acc_ref[0] + ysA_ref[...], acc_ref[1] + ysB_ref[...]], axis=1) * dinv
        h = jnp.dot(agg, w2_ref[...], preferred_element_type=jnp.float32) + b2_ref[...]
        h = _layer_norm_rows(h, g_ref[...], b_ref[...])
        out_ref[...] = (
            jnp.dot(h, wf_ref[...], preferred_element_type=jnp.float32) + bf_ref[...]
        )

    return pl.pallas_call(
        body,
        grid=(_G,),
        in_specs=[
            pl.BlockSpec((NC, _BR, 128), lambda i: (0, i, 0)),
            pl.BlockSpec((_BR, 128), lambda i: (i, 0)),
            pl.BlockSpec((_BR, 128), lambda i: (i, 0)),
            pl.BlockSpec((_BR, 1), lambda i: (i, 0)),
            pl.BlockSpec((256, 256), lambda i: (0, 0)),
            pl.BlockSpec((1, 256), lambda i: (0, 0)),
            pl.BlockSpec((1, 256), lambda i: (0, 0)),
            pl.BlockSpec((1, 256), lambda i: (0, 0)),
            pl.BlockSpec((256, 128), lambda i: (0, 0)),
            pl.BlockSpec((1, 128), lambda i: (0, 0)),
        ],
        out_specs=pl.BlockSpec((_BR, 128), lambda i: (i, 0)),
        out_shape=jax.ShapeDtypeStruct((NP, 128), jnp.float32),
    )(acc2, ysA, ysB, dinv, W2, b2, ln_g, ln_b, Wf, bf)


def kernel(x, edge_index, W1, b1, W2, b2, ln_g, ln_b, Wf, bf):
    src = edge_index[0]
    dst = edge_index[1]
    pad = N + (jnp.arange(EP - E, dtype=jnp.int32) % (NP - N))
    srcp = jnp.concatenate([src, pad])
    dstp = jnp.concatenate([dst, jnp.full((EP - E,), NP - 1, jnp.int32)])
    srcA = srcp.reshape(32, NB1, B, 128)
    dstA = dstp.reshape(32, NB1, B, 128)
    srcB = srcp.reshape(16, NB2, B, 128)
    dstB = dstp.reshape(16, NB2, B, 128)
    x_pad = jnp.pad(x, ((0, NP - N), (0, 0)))

    degp = _sc_deg(dstp.reshape(32, C1, 128))
    xs, dinv = _tc_scale(x_pad, degp)
    acc1 = _sc_spmm1(xs, srcA, dstA)
    b1r = b1.reshape(1, -1)
    gr = ln_g.reshape(1, -1)
    br = ln_b.reshape(1, -1)
    ysA, ysB = _tc_layer1(acc1, xs, dinv, W1, b1r, gr, br)
    acc2 = _sc_spmm2(ysA, ysB, srcB, dstB)
    outp = _tc_layer2(acc2, ysA, ysB, dinv, W2, b2.reshape(1, -1), gr, br,
                      Wf, bf.reshape(1, -1))
    return outp[:N]
